# baseline (speedup 1.0000x reference)
import functools

import jax
import jax.numpy as jnp
from jax import lax
from jax.experimental import pallas as pl
from jax.experimental.pallas import tpu as pltpu
from jax.experimental.pallas import tpu_sc as plsc

EMBED = 128
SROW = 56
CHUNK = 128
NB = 4
GD = 2
SD = 2


@functools.partial(jax.jit, static_argnums=(2, 3, 4))
def _sc_gather(table, idx3, nw, b, s):
    mesh = plsc.VectorSubcoreMesh(core_axis_name="c", subcore_axis_name="s")
    rows_per_w = b // nw
    k_per_w = rows_per_w * SROW // CHUNK
    assert k_per_w % NB == 0 and k_per_w >= NB

    @functools.partial(
        pl.kernel,
        mesh=mesh,
        out_type=jax.ShapeDtypeStruct((b * SROW, EMBED), jnp.float32),
        scratch_types=[
            pltpu.VMEM((k_per_w, CHUNK), jnp.int32),
            pltpu.VMEM((NB, CHUNK, EMBED), jnp.float32),
            pltpu.SemaphoreType.DMA((NB,)),
            pltpu.SemaphoreType.DMA((NB,)),
        ],
    )
    def k(table_hbm, idx_hbm, out_hbm, idx_v, rows_v, gsem, ssem):
        nc = 2
        wid = lax.axis_index("s") * nc + lax.axis_index("c")
        out_base = wid * rows_per_w * SROW
        pltpu.sync_copy(idx_hbm.at[wid], idx_v)

        def gather(j, slot):
            return pltpu.make_async_copy(
                table_hbm.at[idx_v.at[j]], rows_v.at[slot], gsem.at[slot])

        def store(j, slot):
            return pltpu.make_async_copy(
                rows_v.at[slot],
                out_hbm.at[pl.ds(out_base + j * CHUNK, CHUNK)],
                ssem.at[slot])

        for slot in range(GD):
            gather(slot, slot).start()

        def outer(i, _):
            g = i * NB
            for bslot in range(NB):
                j = g + bslot
                nslot = (bslot + GD) % NB
                @pl.when(j + GD - NB >= 0)
                def _():
                    store(j + GD - NB, nslot).wait()

                @pl.when(j + GD < k_per_w)
                def _():
                    gather(j + GD, nslot).start()

                gather(j, bslot).wait()
                store(j, bslot).start()
            return 0

        lax.fori_loop(0, k_per_w // NB, outer, 0)

        for j in range(k_per_w - SD, k_per_w):
            store(j, j % NB).wait()

    return k(table, idx3)


def kernel(cleavage_indices, pos_embed):
    b, s = cleavage_indices.shape
    info = plsc.get_sparse_core_info()
    nw = info.num_cores * info.num_subcores
    rows_per_w = b // nw
    idx = cleavage_indices.astype(jnp.int32).reshape(nw, rows_per_w, s)
    idx = jnp.pad(idx, ((0, 0), (0, 0), (0, SROW - s)))
    idx = idx.reshape(nw, rows_per_w * SROW // CHUNK, CHUNK)
    out = _sc_gather(pos_embed, idx, nw, b, s)
    return out.reshape(b, SROW, EMBED)[:, :s, :]


# speedup vs baseline: 6.4338x; 6.4338x over previous
import functools

import jax
import jax.numpy as jnp
from jax import lax
from jax.experimental import pallas as pl
from jax.experimental.pallas import tpu as pltpu
from jax.experimental.pallas import tpu_sc as plsc

EMBED = 128
SROW = 56
CHUNK = 128
NB = 4
GD = 2
SD = 2


@functools.partial(jax.jit, static_argnums=(2, 3, 4))
def _sc_gather(table, idx3, nw, b, s):
    mesh = plsc.VectorSubcoreMesh(core_axis_name="c", subcore_axis_name="s")
    rows_per_w = b // nw
    k_per_w = rows_per_w * SROW // CHUNK
    assert k_per_w % NB == 0 and k_per_w >= NB

    @functools.partial(
        pl.kernel,
        mesh=mesh,
        out_type=jax.ShapeDtypeStruct((b * SROW, EMBED), jnp.float32),
        scratch_types=[
            pltpu.VMEM((k_per_w, CHUNK), jnp.int32),
            pltpu.VMEM((NB, CHUNK, EMBED), jnp.float32),
            pltpu.SemaphoreType.DMA((NB,)),
            pltpu.SemaphoreType.DMA((NB,)),
        ],
    )
    def k(table_hbm, idx_hbm, out_hbm, idx_v, rows_v, gsem, ssem):
        nc = 2
        wid = lax.axis_index("s") * nc + lax.axis_index("c")
        out_base = wid * rows_per_w * SROW
        pltpu.sync_copy(idx_hbm.at[wid], idx_v)

        def gather(j, slot):
            return pltpu.make_async_copy(
                table_hbm.at[idx_v.at[j]], rows_v.at[slot], gsem.at[slot])

        def store(j, slot):
            return pltpu.make_async_copy(
                rows_v.at[slot],
                out_hbm.at[pl.ds(out_base + j * CHUNK, CHUNK)],
                ssem.at[slot])

        for slot in range(GD):
            gather(slot, slot).start()

        def outer(i, _):
            g = i * NB
            for bslot in range(NB):
                j = g + bslot
                nslot = (bslot + GD) % NB
                @pl.when(j + GD - NB >= 0)
                def _():
                    store(j + GD - NB, nslot).wait()

                @pl.when(j + GD < k_per_w)
                def _():
                    gather(j + GD, nslot).start()

                gather(j, bslot).wait()
                store(j, bslot).start()
            return 0

        lax.fori_loop(0, k_per_w // NB, outer, 0)

        for j in range(k_per_w - SD, k_per_w):
            store(j, j % NB).wait()

    return k(table, idx3)


def kernel(cleavage_indices, pos_embed):
    b, s = cleavage_indices.shape
    info = plsc.get_sparse_core_info()
    nw = info.num_cores * info.num_subcores
    rows_per_w = b // nw
    idx = cleavage_indices.astype(jnp.int32).reshape(nw, rows_per_w, s)
    npad = SROW - s
    v = pos_embed.shape[0]
    pad = (jnp.arange(nw * rows_per_w * npad, dtype=jnp.int32) * 97 % v)
    pad = pad.reshape(nw, rows_per_w, npad)
    idx = jnp.concatenate([idx, pad], axis=2)
    idx = idx.reshape(nw, rows_per_w * SROW // CHUNK, CHUNK)
    out = _sc_gather(pos_embed, idx, nw, b, s)
    return out.reshape(b, SROW, EMBED)[:, :s, :]


# speedup vs baseline: 7.5314x; 1.1706x over previous
import functools

import jax
import jax.numpy as jnp
from jax import lax
from jax.experimental import pallas as pl
from jax.experimental.pallas import tpu as pltpu
from jax.experimental.pallas import tpu_sc as plsc

EMBED = 128
RPC = 2
CPAD = 112
NB = 4
GD = 2
SD = 2


@functools.partial(jax.jit, static_argnums=(2, 3, 4))
def _sc_gather(table, idx3, nw, b, s):
    mesh = plsc.VectorSubcoreMesh(core_axis_name="c", subcore_axis_name="s")
    rows_per_w = b // nw
    k_per_w = rows_per_w // RPC
    assert k_per_w % NB == 0 and k_per_w >= NB

    @functools.partial(
        pl.kernel,
        mesh=mesh,
        out_type=jax.ShapeDtypeStruct((b, s, EMBED), jnp.float32),
        scratch_types=[
            pltpu.VMEM((k_per_w, CPAD), jnp.int32),
            pltpu.VMEM((NB, CPAD, EMBED), jnp.float32),
            pltpu.SemaphoreType.DMA((NB,)),
            pltpu.SemaphoreType.DMA((NB,)),
        ],
    )
    def k(table_hbm, idx_hbm, out_hbm, idx_v, rows_v, gsem, ssem):
        nc = 2
        wid = lax.axis_index("s") * nc + lax.axis_index("c")
        row_base = wid * rows_per_w
        pltpu.sync_copy(idx_hbm.at[wid], idx_v)

        def gather(j, slot):
            return pltpu.make_async_copy(
                table_hbm.at[idx_v.at[j]], rows_v.at[slot], gsem.at[slot])

        def stores(j, slot):
            return [
                pltpu.make_async_copy(
                    rows_v.at[slot, pl.ds(h * s, s)],
                    out_hbm.at[row_base + j * RPC + h],
                    ssem.at[slot])
                for h in range(RPC)
            ]

        for slot in range(GD):
            gather(slot, slot).start()

        def outer(i, _):
            g = i * NB
            for bslot in range(NB):
                j = g + bslot
                nslot = (bslot + GD) % NB
                @pl.when(j + GD - NB >= 0)
                def _():
                    for h_cp in stores(j + GD - NB, nslot):
                        h_cp.wait()

                @pl.when(j + GD < k_per_w)
                def _():
                    gather(j + GD, nslot).start()

                gather(j, bslot).wait()
                for h_cp in stores(j, bslot):
                    h_cp.start()
            return 0

        lax.fori_loop(0, k_per_w // NB, outer, 0)

        for j in range(k_per_w - SD, k_per_w):
            for h_cp in stores(j, j % NB):
                h_cp.wait()

    return k(table, idx3)


def kernel(cleavage_indices, pos_embed):
    b, s = cleavage_indices.shape
    info = plsc.get_sparse_core_info()
    nw = info.num_cores * info.num_subcores
    rows_per_w = b // nw
    k_per_w = rows_per_w // RPC
    idx = cleavage_indices.astype(jnp.int32).reshape(nw, k_per_w, RPC * s)
    npad = CPAD - RPC * s
    v = pos_embed.shape[0]
    pad = (jnp.arange(nw * k_per_w * npad, dtype=jnp.int32) * 97 % v)
    pad = pad.reshape(nw, k_per_w, npad)
    idx = jnp.concatenate([idx, pad], axis=2)
    return _sc_gather(pos_embed, idx, nw, b, s)


# speedup vs baseline: 7.7464x; 1.0286x over previous
import functools

import jax
import jax.numpy as jnp
from jax import lax
from jax.experimental import pallas as pl
from jax.experimental.pallas import tpu as pltpu
from jax.experimental.pallas import tpu_sc as plsc

EMBED = 128
RPC = 2
CPAD = 112
NB = 8
GD = 4
SD = 4


@functools.partial(jax.jit, static_argnums=(2, 3, 4))
def _sc_gather(table, idx3, nw, b, s):
    mesh = plsc.VectorSubcoreMesh(core_axis_name="c", subcore_axis_name="s")
    rows_per_w = b // nw
    k_per_w = rows_per_w // RPC
    assert k_per_w % NB == 0 and k_per_w >= NB

    @functools.partial(
        pl.kernel,
        mesh=mesh,
        out_type=jax.ShapeDtypeStruct((b, s, EMBED), jnp.float32),
        scratch_types=[
            pltpu.VMEM((k_per_w, CPAD), jnp.int32),
            pltpu.VMEM((NB, RPC * s, EMBED), jnp.float32),
            pltpu.SemaphoreType.DMA((NB,)),
            pltpu.SemaphoreType.DMA((NB,)),
        ],
    )
    def k(table_hbm, idx_hbm, out_hbm, idx_v, rows_v, gsem, ssem):
        nc = 2
        wid = lax.axis_index("s") * nc + lax.axis_index("c")
        row_base = wid * rows_per_w
        pltpu.sync_copy(idx_hbm.at[wid], idx_v)

        def gather(j, slot):
            return pltpu.make_async_copy(
                table_hbm.at[idx_v.at[j, pl.ds(0, RPC * s)]],
                rows_v.at[slot], gsem.at[slot])

        def stores(j, slot):
            return [
                pltpu.make_async_copy(
                    rows_v.at[slot, pl.ds(h * s, s)],
                    out_hbm.at[row_base + j * RPC + h],
                    ssem.at[slot])
                for h in range(RPC)
            ]

        for slot in range(GD):
            gather(slot, slot).start()

        def outer(i, _):
            g = i * NB
            for bslot in range(NB):
                j = g + bslot
                nslot = (bslot + GD) % NB
                @pl.when(j + GD - NB >= 0)
                def _():
                    for h_cp in stores(j + GD - NB, nslot):
                        h_cp.wait()

                @pl.when(j + GD < k_per_w)
                def _():
                    gather(j + GD, nslot).start()

                gather(j, bslot).wait()
                for h_cp in stores(j, bslot):
                    h_cp.start()
            return 0

        lax.fori_loop(0, k_per_w // NB, outer, 0)

        for j in range(k_per_w - SD, k_per_w):
            for h_cp in stores(j, j % NB):
                h_cp.wait()

    return k(table, idx3)


def kernel(cleavage_indices, pos_embed):
    b, s = cleavage_indices.shape
    info = plsc.get_sparse_core_info()
    nw = info.num_cores * info.num_subcores
    rows_per_w = b // nw
    k_per_w = rows_per_w // RPC
    idx = cleavage_indices.astype(jnp.int32).reshape(nw, k_per_w, RPC * s)
    npad = CPAD - RPC * s
    v = pos_embed.shape[0]
    pad = (jnp.arange(nw * k_per_w * npad, dtype=jnp.int32) * 97 % v)
    pad = pad.reshape(nw, k_per_w, npad)
    idx = jnp.concatenate([idx, pad], axis=2)
    return _sc_gather(pos_embed, idx, nw, b, s)
